# speedup vs baseline: 2.3801x; 2.3801x over previous
import functools

import jax
import jax.numpy as jnp
from jax import lax
from jax.experimental import pallas as pl
from jax.experimental.pallas import tpu as pltpu
from jax.experimental.pallas import tpu_sc as plsc

HIDDEN = 1024
PATCH = 4
BS = 4
SEQ = 2048

NC = 2
NS = 16
NW = NC * NS

N_ROWS = BS * SEQ * PATCH
ROWS_PER_W = N_ROWS // NW
CHUNK = 32
NCHUNKS = ROWS_PER_W // CHUNK


def _gather_rows(idx, embs):
    mesh = plsc.VectorSubcoreMesh(core_axis_name="c", subcore_axis_name="s")

    @functools.partial(
        pl.kernel,
        out_type=jax.ShapeDtypeStruct((N_ROWS, HIDDEN), jnp.float32),
        mesh=mesh,
        scratch_types=[
            pltpu.VMEM((NCHUNKS, CHUNK), jnp.int32),
            pltpu.VMEM((CHUNK, HIDDEN), jnp.float32),
            pltpu.VMEM((CHUNK, HIDDEN), jnp.float32),
            pltpu.SemaphoreType.DMA,
            pltpu.SemaphoreType.DMA,
            pltpu.SemaphoreType.DMA,
            pltpu.SemaphoreType.DMA,
        ],
    )
    def k(idx_hbm, table_hbm, out_hbm, idx_v, buf0, buf1, sg0, sg1, sw0, sw1):
        wid = lax.axis_index("s") * NC + lax.axis_index("c")
        base = wid * ROWS_PER_W
        pltpu.sync_copy(idx_hbm.at[wid], idx_v)

        def body(i, carry):
            c0 = 2 * i
            r0 = base + c0 * CHUNK
            ga = pltpu.make_async_copy(table_hbm.at[idx_v.at[c0]], buf0, sg0)
            ga.start()
            gb = pltpu.make_async_copy(table_hbm.at[idx_v.at[c0 + 1]], buf1, sg1)
            gb.start()
            ga.wait()
            wa = pltpu.make_async_copy(buf0, out_hbm.at[pl.ds(r0, CHUNK)], sw0)
            wa.start()
            gb.wait()
            wb = pltpu.make_async_copy(buf1, out_hbm.at[pl.ds(r0 + CHUNK, CHUNK)], sw1)
            wb.start()
            wa.wait()
            wb.wait()
            return carry

        lax.fori_loop(0, NCHUNKS // 2, body, 0)

    return k(idx, embs)


MB = 512
K = PATCH * HIDDEN
N_OUT = HIDDEN


def _matmul_kernel(hs_ref, w_ref, o_ref):
    h = hs_ref[...].astype(jnp.bfloat16)
    o_ref[...] = lax.dot_general(
        h, w_ref[...], (((1,), (1,)), ((), ())),
        preferred_element_type=jnp.float32,
    )


def _project(hs, w_bf16):
    m = hs.shape[0]
    return pl.pallas_call(
        _matmul_kernel,
        grid=(m // MB,),
        in_specs=[
            pl.BlockSpec((MB, K), lambda i: (i, 0)),
            pl.BlockSpec((N_OUT, K), lambda i: (0, 0)),
        ],
        out_specs=pl.BlockSpec((MB, N_OUT), lambda i: (i, 0)),
        out_shape=jax.ShapeDtypeStruct((m, N_OUT), jnp.float32),
    )(hs, w_bf16)


def kernel(input_ids, mask, embs, W_proj):
    idx = jnp.where(mask[:, :, None], 0, input_ids + 1).astype(jnp.int32)
    idx = idx.reshape(NW, NCHUNKS, CHUNK)
    hs = _gather_rows(idx, embs)
    hs = hs.reshape(BS * SEQ, K)
    out = _project(hs, W_proj.astype(jnp.bfloat16))
    return out.reshape(BS, SEQ, N_OUT)


# speedup vs baseline: 10.5910x; 4.4499x over previous
import functools

import jax
import jax.numpy as jnp
from jax import lax
from jax.experimental import pallas as pl
from jax.experimental.pallas import tpu as pltpu
from jax.experimental.pallas import tpu_sc as plsc

HIDDEN = 1024
PATCH = 4
BS = 4
SEQ = 2048

NC = 2
NS = 16
NW = NC * NS

N_ROWS = BS * SEQ * PATCH
ROWS_PER_W = N_ROWS // NW
CHUNK = 32
NCHUNKS = ROWS_PER_W // CHUNK


def _gather_rows(idx, embs):
    mesh = plsc.VectorSubcoreMesh(core_axis_name="c", subcore_axis_name="s")

    @functools.partial(
        pl.kernel,
        out_type=jax.ShapeDtypeStruct((N_ROWS, HIDDEN), jnp.float32),
        mesh=mesh,
        scratch_types=[
            pltpu.VMEM((CHUNK,), jnp.int32),
            pltpu.VMEM((CHUNK,), jnp.int32),
            pltpu.VMEM((CHUNK, HIDDEN), jnp.float32),
            pltpu.VMEM((CHUNK, HIDDEN), jnp.float32),
            pltpu.SemaphoreType.DMA,
            pltpu.SemaphoreType.DMA,
            pltpu.SemaphoreType.DMA,
            pltpu.SemaphoreType.DMA,
        ],
    )
    def k(idx_hbm, table_hbm, out_hbm, idx0, idx1, buf0, buf1,
          sg0, sg1, sw0, sw1):
        wid = lax.axis_index("s") * NC + lax.axis_index("c")
        base = wid * ROWS_PER_W
        pltpu.sync_copy(idx_hbm.at[wid, 0], idx0)

        def body(i, carry):
            c0 = 2 * i
            r0 = base + c0 * CHUNK
            ga = pltpu.make_async_copy(table_hbm.at[idx0], buf0, sg0)
            ga.start()
            pltpu.sync_copy(idx_hbm.at[wid, c0 + 1], idx1)
            gb = pltpu.make_async_copy(table_hbm.at[idx1], buf1, sg1)
            gb.start()
            ga.wait()
            wa = pltpu.make_async_copy(buf0, out_hbm.at[pl.ds(r0, CHUNK)], sw0)
            wa.start()
            nxt = jnp.minimum(c0 + 2, NCHUNKS - 1)
            pltpu.sync_copy(idx_hbm.at[wid, nxt], idx0)
            gb.wait()
            wb = pltpu.make_async_copy(buf1, out_hbm.at[pl.ds(r0 + CHUNK, CHUNK)], sw1)
            wb.start()
            wa.wait()
            wb.wait()
            return carry

        lax.fori_loop(0, NCHUNKS // 2, body, 0)

    return k(idx, embs)


MB = 512
K = PATCH * HIDDEN
N_OUT = HIDDEN


def _matmul_kernel(hs_ref, w_ref, e0_ref, m_ref, o_ref):
    h = hs_ref[...].astype(jnp.bfloat16)
    out = lax.dot_general(
        h, w_ref[...], (((1,), (1,)), ((), ())),
        preferred_element_type=jnp.float32,
    )
    e0 = jnp.concatenate([e0_ref[...]] * PATCH, axis=1).astype(jnp.bfloat16)
    v0 = lax.dot_general(
        e0, w_ref[...], (((1,), (1,)), ((), ())),
        preferred_element_type=jnp.float32,
    )
    o_ref[...] = jnp.where(m_ref[...] != 0, v0, out)


def _project(hs, w_bf16, e0, mask2):
    m = hs.shape[0]
    return pl.pallas_call(
        _matmul_kernel,
        grid=(m // MB,),
        in_specs=[
            pl.BlockSpec((MB, K), lambda i: (i, 0)),
            pl.BlockSpec((N_OUT, K), lambda i: (0, 0)),
            pl.BlockSpec((1, HIDDEN), lambda i: (0, 0)),
            pl.BlockSpec((MB, 1), lambda i: (i, 0)),
        ],
        out_specs=pl.BlockSpec((MB, N_OUT), lambda i: (i, 0)),
        out_shape=jax.ShapeDtypeStruct((m, N_OUT), jnp.float32),
    )(hs, w_bf16, e0, mask2)


def kernel(input_ids, mask, embs, W_proj):
    idx = (input_ids.astype(jnp.int32) + 1).reshape(NW, NCHUNKS, CHUNK)
    hs = _gather_rows(idx, embs)
    hs = hs.reshape(BS * SEQ, K)
    mask2 = mask.reshape(BS * SEQ, 1).astype(jnp.int32)
    out = _project(hs, W_proj.astype(jnp.bfloat16), embs[0:1], mask2)
    return out.reshape(BS, SEQ, N_OUT)


# speedup vs baseline: 18.4358x; 1.7407x over previous
import functools

import jax
import jax.numpy as jnp
from jax import lax
from jax.experimental import pallas as pl
from jax.experimental.pallas import tpu as pltpu
from jax.experimental.pallas import tpu_sc as plsc

HIDDEN = 1024
PATCH = 4
BS = 4
SEQ = 2048

NC = 2
NS = 16
NW = NC * NS

N_ROWS = BS * SEQ * PATCH
ROWS_PER_W = N_ROWS // NW
CHUNK = 32
NCHUNKS = ROWS_PER_W // CHUNK


def _gather_rows(idx, embs):
    mesh = plsc.VectorSubcoreMesh(core_axis_name="c", subcore_axis_name="s")

    @functools.partial(
        pl.kernel,
        out_type=jax.ShapeDtypeStruct((N_ROWS, HIDDEN), jnp.float32),
        mesh=mesh,
        scratch_types=[
            pltpu.VMEM((CHUNK,), jnp.int32),
            pltpu.VMEM((CHUNK,), jnp.int32),
            pltpu.VMEM((CHUNK, HIDDEN), jnp.float32),
            pltpu.VMEM((CHUNK, HIDDEN), jnp.float32),
            pltpu.SemaphoreType.DMA,
            pltpu.SemaphoreType.DMA,
            pltpu.SemaphoreType.DMA,
            pltpu.SemaphoreType.DMA,
        ],
    )
    def k(idx_hbm, table_hbm, out_hbm, idx0, idx1, buf0, buf1,
          sg0, sg1, sw0, sw1):
        wid = lax.axis_index("s") * NC + lax.axis_index("c")
        base = wid * ROWS_PER_W
        pltpu.sync_copy(idx_hbm.at[wid, 0], idx0)

        def body(i, carry):
            c0 = 2 * i
            r0 = base + c0 * CHUNK
            ga = pltpu.make_async_copy(table_hbm.at[idx0], buf0, sg0)
            ga.start()
            pltpu.sync_copy(idx_hbm.at[wid, c0 + 1], idx1)
            gb = pltpu.make_async_copy(table_hbm.at[idx1], buf1, sg1)
            gb.start()
            ga.wait()
            wa = pltpu.make_async_copy(buf0, out_hbm.at[pl.ds(r0, CHUNK)], sw0)
            wa.start()
            nxt = jnp.minimum(c0 + 2, NCHUNKS - 1)
            pltpu.sync_copy(idx_hbm.at[wid, nxt], idx0)
            gb.wait()
            wb = pltpu.make_async_copy(buf1, out_hbm.at[pl.ds(r0 + CHUNK, CHUNK)], sw1)
            wb.start()
            wa.wait()
            wb.wait()
            return carry

        lax.fori_loop(0, NCHUNKS // 2, body, 0)

    return k(idx, embs)


MB = 512
K = PATCH * HIDDEN
N_OUT = HIDDEN


def _matmul_kernel(hs_ref, w_ref, e0_ref, m_ref, o_ref):
    acc = lax.dot_general(
        hs_ref[0].astype(jnp.bfloat16), w_ref[0],
        (((1,), (0,)), ((), ())), preferred_element_type=jnp.float32,
    )
    for p in range(1, PATCH):
        acc += lax.dot_general(
            hs_ref[p].astype(jnp.bfloat16), w_ref[p],
            (((1,), (0,)), ((), ())), preferred_element_type=jnp.float32,
        )
    wsum = w_ref[0] + w_ref[1] + w_ref[2] + w_ref[3]
    v0 = lax.dot_general(
        e0_ref[...].astype(jnp.bfloat16), wsum,
        (((1,), (0,)), ((), ())), preferred_element_type=jnp.float32,
    )
    o_ref[...] = jnp.where(m_ref[...] != 0, v0, acc)


def _project(hs_p, w_r, e0, mask2):
    m = hs_p.shape[1]
    return pl.pallas_call(
        _matmul_kernel,
        grid=(m // MB,),
        in_specs=[
            pl.BlockSpec((PATCH, MB, HIDDEN), lambda i: (0, i, 0)),
            pl.BlockSpec((PATCH, HIDDEN, N_OUT), lambda i: (0, 0, 0)),
            pl.BlockSpec((1, HIDDEN), lambda i: (0, 0)),
            pl.BlockSpec((MB, 1), lambda i: (i, 0)),
        ],
        out_specs=pl.BlockSpec((MB, N_OUT), lambda i: (i, 0)),
        out_shape=jax.ShapeDtypeStruct((m, N_OUT), jnp.float32),
    )(hs_p, w_r, e0, mask2)


def kernel(input_ids, mask, embs, W_proj):
    idx = jnp.transpose(input_ids.astype(jnp.int32) + 1, (2, 0, 1))
    idx = idx.reshape(NW, NCHUNKS, CHUNK)
    hs = _gather_rows(idx, embs)
    hs_p = hs.reshape(PATCH, BS * SEQ, HIDDEN)
    w_r = jnp.transpose(
        W_proj.reshape(N_OUT, PATCH, HIDDEN), (1, 2, 0)
    ).astype(jnp.bfloat16)
    mask2 = mask.reshape(BS * SEQ, 1).astype(jnp.int32)
    out = _project(hs_p, w_r, embs[0:1], mask2)
    return out.reshape(BS, SEQ, N_OUT)
